# baseline (speedup 1.0000x reference)
import jax
import jax.numpy as jnp
from jax import lax
from jax.experimental import pallas as pl
from jax.experimental.pallas import tpu as pltpu
from jax.experimental.pallas import tpu_sc as plsc

ROWS = 16384
COLS = 256
NC = 2
NS = 16
L = 16
NW = NC * NS
RPW = ROWS // NW
CHUNK = 64
NCHUNK = RPW // CHUNK
GROUPS = COLS // L


def _permute_body(x_hbm, perm_hbm, out_hbm, perm_v, xin_v, out_v):
    wid = lax.axis_index("s") * NC + lax.axis_index("c")
    base = wid * (RPW * COLS)

    pltpu.sync_copy(perm_hbm, perm_v)

    def do_chunk(c, _):
        off = base + c * (CHUNK * COLS)
        pltpu.sync_copy(x_hbm.at[pl.ds(off, CHUNK * COLS)], xin_v)

        idx0 = [perm_v[pl.ds(g * L, L)] for g in range(GROUPS)]

        def do_row(r, idx):
            robase = r * COLS
            for g in range(GROUPS):
                vals = plsc.load_gather(xin_v, [idx[g]])
                out_v[pl.ds(robase + g * L, L)] = vals
            return [i + COLS for i in idx]

        lax.fori_loop(0, CHUNK, do_row, idx0, unroll=False)
        pltpu.sync_copy(out_v, out_hbm.at[pl.ds(off, CHUNK * COLS)])
        return 0

    lax.fori_loop(0, NCHUNK, do_chunk, 0, unroll=False)


@jax.jit
def kernel(x, permutation):
    mesh = plsc.VectorSubcoreMesh(core_axis_name="c", subcore_axis_name="s")
    run = pl.kernel(
        _permute_body,
        mesh=mesh,
        out_type=jax.ShapeDtypeStruct((ROWS * COLS,), jnp.float32),
        compiler_params=pltpu.CompilerParams(needs_layout_passes=False),
        scratch_types=[
            pltpu.VMEM((COLS,), jnp.int32),
            pltpu.VMEM((CHUNK * COLS,), jnp.float32),
            pltpu.VMEM((CHUNK * COLS,), jnp.float32),
        ],
    )
    y_flat = run(x.reshape(-1), permutation)
    return y_flat.reshape(ROWS, COLS)


# speedup vs baseline: 1.1143x; 1.1143x over previous
import jax
import jax.numpy as jnp
from jax import lax
from jax.experimental import pallas as pl
from jax.experimental.pallas import tpu as pltpu
from jax.experimental.pallas import tpu_sc as plsc

ROWS = 16384
COLS = 256
NC = 2
NS = 16
L = 16
NW = NC * NS
RPW = ROWS // NW
CHUNK = 64
NCHUNK = RPW // CHUNK
GROUPS = COLS // L


def _permute_body(x_hbm, perm_hbm, out_hbm, perm_v,
                  xin0, xin1, out0, out1, isem0, isem1, osem0, osem1):
    wid = lax.axis_index("s") * NC + lax.axis_index("c")
    base = wid * (RPW * COLS)

    pltpu.sync_copy(perm_hbm, perm_v)
    idx0 = [perm_v[pl.ds(g * L, L)] for g in range(GROUPS)]

    xin = [xin0, xin1]
    xout = [out0, out1]
    isem = [isem0, isem1]
    osem = [osem0, osem1]

    def start_in(c):
        off = base + c * (CHUNK * COLS)
        return pltpu.make_async_copy(
            x_hbm.at[pl.ds(off, CHUNK * COLS)], xin[c % 2], isem[c % 2]
        )

    def start_out(c):
        off = base + c * (CHUNK * COLS)
        return pltpu.make_async_copy(
            xout[c % 2], out_hbm.at[pl.ds(off, CHUNK * COLS)], osem[c % 2]
        )

    def compute(c):
        src = xin[c % 2]
        dst = xout[c % 2]

        def do_row(r, idx):
            robase = r * COLS
            for g in range(GROUPS):
                dst[pl.ds(robase + g * L, L)] = plsc.load_gather(src, [idx[g]])
            return [i + COLS for i in idx]

        lax.fori_loop(0, CHUNK, do_row, idx0, unroll=False)

    in_cp = [None] * NCHUNK
    out_cp = [None] * NCHUNK
    in_cp[0] = start_in(0)
    in_cp[0].start()
    for c in range(NCHUNK):
        if c + 1 < NCHUNK:
            in_cp[c + 1] = start_in(c + 1)
            in_cp[c + 1].start()
        in_cp[c].wait()
        if c >= 2:
            out_cp[c - 2].wait()
        compute(c)
        out_cp[c] = start_out(c)
        out_cp[c].start()
    out_cp[NCHUNK - 2].wait()
    out_cp[NCHUNK - 1].wait()


@jax.jit
def kernel(x, permutation):
    mesh = plsc.VectorSubcoreMesh(core_axis_name="c", subcore_axis_name="s")
    run = pl.kernel(
        _permute_body,
        mesh=mesh,
        out_type=jax.ShapeDtypeStruct((ROWS * COLS,), jnp.float32),
        compiler_params=pltpu.CompilerParams(needs_layout_passes=False),
        scratch_types=[
            pltpu.VMEM((COLS,), jnp.int32),
            pltpu.VMEM((CHUNK * COLS,), jnp.float32),
            pltpu.VMEM((CHUNK * COLS,), jnp.float32),
            pltpu.VMEM((CHUNK * COLS,), jnp.float32),
            pltpu.VMEM((CHUNK * COLS,), jnp.float32),
            pltpu.SemaphoreType.DMA,
            pltpu.SemaphoreType.DMA,
            pltpu.SemaphoreType.DMA,
            pltpu.SemaphoreType.DMA,
        ],
    )
    y_flat = run(x.reshape(-1), permutation)
    return y_flat.reshape(ROWS, COLS)


# speedup vs baseline: 1.4532x; 1.3041x over previous
import jax
import jax.numpy as jnp
from jax import lax
from jax.experimental import pallas as pl
from jax.experimental.pallas import tpu as pltpu
from jax.experimental.pallas import tpu_sc as plsc

ROWS = 16384
COLS = 256
NC = 2
NS = 16
L = 16
NW = NC * NS
RPW = ROWS // NW
CHUNK = 64
NCHUNK = RPW // CHUNK
GROUPS = COLS // L


def _permute_body(x_hbm, perm_hbm, out_hbm, perm_v,
                  xin0, xin1, out0, out1, isem0, isem1, osem0, osem1):
    wid = lax.axis_index("s") * NC + lax.axis_index("c")
    base = wid * (RPW * COLS)

    pltpu.sync_copy(perm_hbm, perm_v)
    idx0 = [perm_v[pl.ds(g * L, L)] for g in range(GROUPS)]

    xin = [xin0, xin1]
    xout = [out0, out1]
    isem = [isem0, isem1]
    osem = [osem0, osem1]

    def start_in(c):
        off = base + c * (CHUNK * COLS)
        return pltpu.make_async_copy(
            x_hbm.at[pl.ds(off, CHUNK * COLS)], xin[c % 2], isem[c % 2]
        )

    def start_out(c):
        off = base + c * (CHUNK * COLS)
        return pltpu.make_async_copy(
            xout[c % 2], out_hbm.at[pl.ds(off, CHUNK * COLS)], osem[c % 2]
        )

    def compute(c):
        src = xin[c % 2]
        dst = xout[c % 2]

        @plsc.parallel_loop(0, CHUNK, unroll=4)
        def do_row(r):
            robase = r * COLS
            vals = [plsc.load_gather(src, [idx0[g] + robase])
                    for g in range(GROUPS)]
            for g in range(GROUPS):
                dst[pl.ds(robase + g * L, L)] = vals[g]

    in_cp = [None] * NCHUNK
    out_cp = [None] * NCHUNK
    in_cp[0] = start_in(0)
    in_cp[0].start()
    for c in range(NCHUNK):
        if c + 1 < NCHUNK:
            in_cp[c + 1] = start_in(c + 1)
            in_cp[c + 1].start()
        in_cp[c].wait()
        if c >= 2:
            out_cp[c - 2].wait()
        compute(c)
        out_cp[c] = start_out(c)
        out_cp[c].start()
    out_cp[NCHUNK - 2].wait()
    out_cp[NCHUNK - 1].wait()


@jax.jit
def kernel(x, permutation):
    mesh = plsc.VectorSubcoreMesh(core_axis_name="c", subcore_axis_name="s")
    run = pl.kernel(
        _permute_body,
        mesh=mesh,
        out_type=jax.ShapeDtypeStruct((ROWS * COLS,), jnp.float32),
        compiler_params=pltpu.CompilerParams(needs_layout_passes=False),
        scratch_types=[
            pltpu.VMEM((COLS,), jnp.int32),
            pltpu.VMEM((CHUNK * COLS,), jnp.float32),
            pltpu.VMEM((CHUNK * COLS,), jnp.float32),
            pltpu.VMEM((CHUNK * COLS,), jnp.float32),
            pltpu.VMEM((CHUNK * COLS,), jnp.float32),
            pltpu.SemaphoreType.DMA,
            pltpu.SemaphoreType.DMA,
            pltpu.SemaphoreType.DMA,
            pltpu.SemaphoreType.DMA,
        ],
    )
    y_flat = run(x.reshape(-1), permutation)
    return y_flat.reshape(ROWS, COLS)


# speedup vs baseline: 2.2529x; 1.5503x over previous
import jax
import jax.numpy as jnp
from jax import lax
from jax.experimental import pallas as pl
from jax.experimental.pallas import tpu as pltpu
from jax.experimental.pallas import tpu_sc as plsc

ROWS = 16384
COLS = 256
NC = 2
NS = 16
L = 16
NW = NC * NS
RPW = ROWS // NW
CHUNK = 64
NCHUNK = RPW // CHUNK
GROUPS = COLS // L


def _permute_body(x_hbm, perm_hbm, out_hbm, perm_v,
                  xin0, xin1, out0, out1, isem0, isem1, osem0, osem1):
    wid = lax.axis_index("s") * NC + lax.axis_index("c")
    row_base = wid * RPW

    pltpu.sync_copy(perm_hbm, perm_v)
    idx0 = [perm_v[pl.ds(g * L, L)] for g in range(GROUPS)]

    xin = [xin0, xin1]
    xout = [out0, out1]
    isem = [isem0, isem1]
    osem = [osem0, osem1]

    def start_in(c):
        r0 = row_base + c * CHUNK
        return pltpu.make_async_copy(
            x_hbm.at[pl.ds(r0, CHUNK), :], xin[c % 2], isem[c % 2]
        )

    def start_out(c):
        r0 = row_base + c * CHUNK
        return pltpu.make_async_copy(
            xout[c % 2], out_hbm.at[pl.ds(r0, CHUNK), :], osem[c % 2]
        )

    def compute(c):
        src = xin[c % 2]
        dst = xout[c % 2]

        @plsc.parallel_loop(0, CHUNK, unroll=4)
        def do_row(r):
            rvec = jnp.full((L,), r, dtype=jnp.int32)
            vals = [plsc.load_gather(src, [rvec, idx0[g]])
                    for g in range(GROUPS)]
            for g in range(GROUPS):
                dst[r, pl.ds(g * L, L)] = vals[g]

    in_cp = [None] * NCHUNK
    out_cp = [None] * NCHUNK
    in_cp[0] = start_in(0)
    in_cp[0].start()
    for c in range(NCHUNK):
        if c + 1 < NCHUNK:
            in_cp[c + 1] = start_in(c + 1)
            in_cp[c + 1].start()
        in_cp[c].wait()
        if c >= 2:
            out_cp[c - 2].wait()
        compute(c)
        out_cp[c] = start_out(c)
        out_cp[c].start()
    out_cp[NCHUNK - 2].wait()
    out_cp[NCHUNK - 1].wait()


@jax.jit
def kernel(x, permutation):
    mesh = plsc.VectorSubcoreMesh(core_axis_name="c", subcore_axis_name="s")
    run = pl.kernel(
        _permute_body,
        mesh=mesh,
        out_type=jax.ShapeDtypeStruct((ROWS, COLS), jnp.float32),
        compiler_params=pltpu.CompilerParams(needs_layout_passes=False),
        scratch_types=[
            pltpu.VMEM((COLS,), jnp.int32),
            pltpu.VMEM((CHUNK, COLS), jnp.float32),
            pltpu.VMEM((CHUNK, COLS), jnp.float32),
            pltpu.VMEM((CHUNK, COLS), jnp.float32),
            pltpu.VMEM((CHUNK, COLS), jnp.float32),
            pltpu.SemaphoreType.DMA,
            pltpu.SemaphoreType.DMA,
            pltpu.SemaphoreType.DMA,
            pltpu.SemaphoreType.DMA,
        ],
    )
    return run(x, permutation)


# speedup vs baseline: 2.8011x; 1.2434x over previous
import jax
import jax.numpy as jnp
from jax import lax
from jax.experimental import pallas as pl
from jax.experimental.pallas import tpu as pltpu
from jax.experimental.pallas import tpu_sc as plsc

ROWS = 16384
COLS = 256
NC = 2
NS = 16
L = 16
NW = NC * NS
RPW = ROWS // NW
CHUNK = 64
NCHUNK = RPW // CHUNK
GROUPS = COLS // L


def _permute_body(x_hbm, perm_hbm, out_hbm, perm_v,
                  xin0, xin1, out0, out1, isem0, isem1, osem0, osem1):
    wid = lax.axis_index("s") * NC + lax.axis_index("c")
    row_base = wid * RPW

    pltpu.sync_copy(perm_hbm, perm_v)
    idx0 = [perm_v[pl.ds(g * L, L)] for g in range(GROUPS)]

    xin = [xin0, xin1]
    xout = [out0, out1]
    isem = [isem0, isem1]
    osem = [osem0, osem1]

    def start_in(c):
        r0 = row_base + c * CHUNK
        return pltpu.make_async_copy(
            x_hbm.at[pl.ds(r0, CHUNK), :], xin[c % 2], isem[c % 2]
        )

    def start_out(c):
        r0 = row_base + c * CHUNK
        return pltpu.make_async_copy(
            xout[c % 2], out_hbm.at[pl.ds(r0, CHUNK), :], osem[c % 2]
        )

    def compute(c):
        src = xin[c % 2]
        dst = xout[c % 2]

        @plsc.parallel_loop(0, CHUNK, unroll=2)
        def do_row(r):
            rvec = jnp.full((L,), r, dtype=jnp.int32)
            for g in range(GROUPS):
                dst[r, pl.ds(g * L, L)] = plsc.load_gather(
                    src, [rvec, idx0[g]])

    in_cp = [None] * NCHUNK
    out_cp = [None] * NCHUNK
    in_cp[0] = start_in(0)
    in_cp[0].start()
    for c in range(NCHUNK):
        if c + 1 < NCHUNK:
            in_cp[c + 1] = start_in(c + 1)
            in_cp[c + 1].start()
        in_cp[c].wait()
        if c >= 2:
            out_cp[c - 2].wait()
        compute(c)
        out_cp[c] = start_out(c)
        out_cp[c].start()
    out_cp[NCHUNK - 2].wait()
    out_cp[NCHUNK - 1].wait()


@jax.jit
def kernel(x, permutation):
    mesh = plsc.VectorSubcoreMesh(core_axis_name="c", subcore_axis_name="s")
    run = pl.kernel(
        _permute_body,
        mesh=mesh,
        out_type=jax.ShapeDtypeStruct((ROWS, COLS), jnp.float32),
        compiler_params=pltpu.CompilerParams(needs_layout_passes=False),
        scratch_types=[
            pltpu.VMEM((COLS,), jnp.int32),
            pltpu.VMEM((CHUNK, COLS), jnp.float32),
            pltpu.VMEM((CHUNK, COLS), jnp.float32),
            pltpu.VMEM((CHUNK, COLS), jnp.float32),
            pltpu.VMEM((CHUNK, COLS), jnp.float32),
            pltpu.SemaphoreType.DMA,
            pltpu.SemaphoreType.DMA,
            pltpu.SemaphoreType.DMA,
            pltpu.SemaphoreType.DMA,
        ],
    )
    return run(x, permutation)


# speedup vs baseline: 3.1167x; 1.1127x over previous
import jax
import jax.numpy as jnp
from jax import lax
from jax.experimental import pallas as pl
from jax.experimental.pallas import tpu as pltpu
from jax.experimental.pallas import tpu_sc as plsc

ROWS = 16384
COLS = 256
NC = 2
NS = 16
L = 16
NW = NC * NS
RPW = ROWS // NW
CHUNK = 64
NCHUNK = RPW // CHUNK
GROUPS = COLS // L


def _permute_body(x_hbm, perm_hbm, out_hbm, perm_v,
                  xin0, xin1, out0, out1, isem0, isem1, osem0, osem1):
    wid = lax.axis_index("s") * NC + lax.axis_index("c")
    row_base = wid * RPW

    pltpu.sync_copy(perm_hbm, perm_v)
    idx0 = [perm_v[pl.ds(g * L, L)] for g in range(GROUPS)]

    xin = [xin0, xin1]
    xout = [out0, out1]
    isem = [isem0, isem1]
    osem = [osem0, osem1]

    def in_copy(c, b):
        r0 = row_base + c * CHUNK
        return pltpu.make_async_copy(
            x_hbm.at[pl.ds(r0, CHUNK), :], xin[b], isem[b]
        )

    def out_copy(c, b):
        r0 = row_base + c * CHUNK
        return pltpu.make_async_copy(
            xout[b], out_hbm.at[pl.ds(r0, CHUNK), :], osem[b]
        )

    def compute(b):
        src = xin[b]
        dst = xout[b]

        @plsc.parallel_loop(0, CHUNK, unroll=2)
        def do_row(r):
            rvec = jnp.full((L,), r, dtype=jnp.int32)
            for g in range(GROUPS):
                dst[r, pl.ds(g * L, L)] = plsc.load_gather(
                    src, [rvec, idx0[g]])

    in_copy(0, 0).start()
    in_copy(1, 1).start()

    def ring_body(i, _):
        for b in range(2):
            c = i * 2 + b
            in_copy(c, b).wait()

            @pl.when(c + 2 < NCHUNK)
            def _():
                in_copy(c + 2, b).start()

            @pl.when(c >= 2)
            def _():
                out_copy(c - 2, b).wait()

            compute(b)
            out_copy(c, b).start()
        return 0

    lax.fori_loop(0, NCHUNK // 2, ring_body, 0, unroll=False)
    out_copy(NCHUNK - 2, 0).wait()
    out_copy(NCHUNK - 1, 1).wait()


@jax.jit
def kernel(x, permutation):
    mesh = plsc.VectorSubcoreMesh(core_axis_name="c", subcore_axis_name="s")
    run = pl.kernel(
        _permute_body,
        mesh=mesh,
        out_type=jax.ShapeDtypeStruct((ROWS, COLS), jnp.float32),
        compiler_params=pltpu.CompilerParams(needs_layout_passes=False),
        scratch_types=[
            pltpu.VMEM((COLS,), jnp.int32),
            pltpu.VMEM((CHUNK, COLS), jnp.float32),
            pltpu.VMEM((CHUNK, COLS), jnp.float32),
            pltpu.VMEM((CHUNK, COLS), jnp.float32),
            pltpu.VMEM((CHUNK, COLS), jnp.float32),
            pltpu.SemaphoreType.DMA,
            pltpu.SemaphoreType.DMA,
            pltpu.SemaphoreType.DMA,
            pltpu.SemaphoreType.DMA,
        ],
    )
    return run(x, permutation)
